# baseline (speedup 1.0000x reference)
import functools

import jax
import jax.numpy as jnp
from jax import lax
from jax.experimental import pallas as pl
from jax.experimental.pallas import tpu as pltpu
from jax.experimental.pallas import tpu_sc as plsc

VOCAB = 1000
N_TOKENS = 1024 * 200
NUM_CORES = 2
NUM_SUBCORES = 16
NUM_WORKERS = NUM_CORES * NUM_SUBCORES
B_PER_W = N_TOKENS // NUM_WORKERS
CHUNK = 128
N_CHUNKS = B_PER_W // CHUNK


@functools.partial(
    pl.kernel,
    out_type=jax.ShapeDtypeStruct((N_TOKENS, VOCAB), jnp.float32),
    mesh=plsc.VectorSubcoreMesh(core_axis_name="c", subcore_axis_name="s"),
    compiler_params=pltpu.CompilerParams(use_tc_tiling_on_sc=False),
    scratch_types=[
        pltpu.VMEM((CHUNK,), jnp.int32),
        pltpu.VMEM((CHUNK, VOCAB), jnp.float32),
        pltpu.SemaphoreType.DMA,
    ],
)
def _gather_kernel(idx_hbm, table_hbm, out_hbm, idx_v, rows_v, sem):
    wid = lax.axis_index("s") * NUM_CORES + lax.axis_index("c")
    base = wid * B_PER_W

    def body(i, carry):
        off = base + i * CHUNK
        pltpu.sync_copy(idx_hbm.at[pl.ds(off, CHUNK)], idx_v)
        pltpu.async_copy(table_hbm.at[idx_v], rows_v, sem).wait()
        pltpu.sync_copy(rows_v, out_hbm.at[pl.ds(off, CHUNK)])
        return carry

    lax.fori_loop(0, N_CHUNKS, body, 0)


def kernel(inputs, embedding):
    idx = inputs.reshape(-1).astype(jnp.int32)
    out = _gather_kernel(idx, embedding)
    return out.reshape(inputs.shape[0], inputs.shape[1], VOCAB)


# speedup vs baseline: 1.4443x; 1.4443x over previous
import functools

import jax
import jax.numpy as jnp
from jax import lax
from jax.experimental import pallas as pl
from jax.experimental.pallas import tpu as pltpu
from jax.experimental.pallas import tpu_sc as plsc

VOCAB = 1000
VOCAB_PAD = 1024
TAIL_START = 896
TAIL = VOCAB - TAIL_START
N_TOKENS = 1024 * 200
NUM_CORES = 2
NUM_SUBCORES = 16
NUM_WORKERS = NUM_CORES * NUM_SUBCORES
B_PER_W = N_TOKENS // NUM_WORKERS
CHUNK = 64
N_CHUNKS = B_PER_W // CHUNK


@functools.partial(
    pl.kernel,
    out_type=jax.ShapeDtypeStruct((N_TOKENS, VOCAB), jnp.float32),
    mesh=plsc.VectorSubcoreMesh(core_axis_name="c", subcore_axis_name="s"),
    scratch_types=[
        pltpu.VMEM((CHUNK,), jnp.int32),
        pltpu.VMEM((CHUNK, VOCAB_PAD), jnp.float32),
        pltpu.VMEM((CHUNK, TAIL), jnp.float32),
        pltpu.SemaphoreType.DMA,
    ],
)
def _gather_kernel(idx_hbm, table_hbm, out_hbm, idx_v, rows_v, tail_v, sem):
    wid = lax.axis_index("s") * NUM_CORES + lax.axis_index("c")
    base = wid * B_PER_W

    def body(i, carry):
        off = base + i * CHUNK
        pltpu.sync_copy(idx_hbm.at[pl.ds(off, CHUNK)], idx_v)
        pltpu.async_copy(table_hbm.at[idx_v], rows_v, sem).wait()
        for k in range(7):
            pltpu.sync_copy(
                rows_v.at[:, pl.ds(128 * k, 128)],
                out_hbm.at[pl.ds(off, CHUNK), pl.ds(128 * k, 128)])

        def repack_row(r, c2):
            for t in range(6):
                tail_v[r, pl.ds(16 * t, 16)] = rows_v[r, pl.ds(TAIL_START + 16 * t, 16)]
            tail_v[r, pl.ds(TAIL - 16, 16)] = rows_v[r, pl.ds(VOCAB - 16, 16)]
            return c2

        lax.fori_loop(0, CHUNK, repack_row, 0)
        pltpu.sync_copy(tail_v,
                        out_hbm.at[pl.ds(off, CHUNK), pl.ds(TAIL_START, TAIL)])
        return carry

    lax.fori_loop(0, N_CHUNKS, body, 0)


def kernel(inputs, embedding):
    idx = inputs.reshape(-1).astype(jnp.int32)
    table = jnp.pad(embedding, ((0, 0), (0, VOCAB_PAD - VOCAB)))
    out = _gather_kernel(idx, table)
    return out.reshape(inputs.shape[0], inputs.shape[1], VOCAB)


# speedup vs baseline: 1.7273x; 1.1959x over previous
import functools

import jax
import jax.numpy as jnp
from jax import lax
from jax.experimental import pallas as pl
from jax.experimental.pallas import tpu as pltpu
from jax.experimental.pallas import tpu_sc as plsc

VOCAB = 1000
VOCAB_PAD = 1024
TAIL_START = 896
TAIL = VOCAB - TAIL_START
N_TOKENS = 1024 * 200
NUM_CORES = 2
NUM_SUBCORES = 16
NUM_WORKERS = NUM_CORES * NUM_SUBCORES
B_PER_W = N_TOKENS // NUM_WORKERS
CHUNK = 40
N_CHUNKS = B_PER_W // CHUNK
NBUF = 2


@functools.partial(
    pl.kernel,
    out_type=jax.ShapeDtypeStruct((N_TOKENS, VOCAB), jnp.float32),
    mesh=plsc.VectorSubcoreMesh(core_axis_name="c", subcore_axis_name="s"),
    scratch_types=[
        pltpu.VMEM((B_PER_W,), jnp.int32),
        [pltpu.VMEM((CHUNK, VOCAB_PAD), jnp.float32) for _ in range(NBUF)],
        [pltpu.VMEM((CHUNK, TAIL), jnp.float32) for _ in range(NBUF)],
        [pltpu.SemaphoreType.DMA for _ in range(NBUF)],
        [pltpu.SemaphoreType.DMA for _ in range(NBUF)],
        [pltpu.SemaphoreType.DMA for _ in range(NBUF)],
    ],
)
def _gather_kernel(idx_hbm, table_hbm, out_hbm, idx_v, rows_v, tail_v,
                   sem_g, sem_b, sem_t):
    wid = lax.axis_index("s") * NUM_CORES + lax.axis_index("c")
    base = wid * B_PER_W
    pltpu.sync_copy(idx_hbm.at[pl.ds(base, B_PER_W)], idx_v)

    def start_gather(i, b):
        pltpu.async_copy(
            table_hbm.at[idx_v.at[pl.ds(i * CHUNK, CHUNK)]], rows_v[b],
            sem_g[b])

    def finish_chunk(i, b):
        pltpu.make_async_copy(
            table_hbm.at[idx_v.at[pl.ds(i * CHUNK, CHUNK)]], rows_v[b],
            sem_g[b]).wait()
        off = base + i * CHUNK
        pltpu.async_copy(
            rows_v[b].at[:, pl.ds(0, TAIL_START)],
            out_hbm.at[pl.ds(off, CHUNK), pl.ds(0, TAIL_START)], sem_b[b])

        def repack_row(r, c):
            for t in range(6):
                tail_v[b][r, pl.ds(16 * t, 16)] = (
                    rows_v[b][r, pl.ds(TAIL_START + 16 * t, 16)])
            tail_v[b][r, pl.ds(TAIL - 16, 16)] = (
                rows_v[b][r, pl.ds(VOCAB - 16, 16)])
            return c

        lax.fori_loop(0, CHUNK, repack_row, 0)
        pltpu.async_copy(
            tail_v[b],
            out_hbm.at[pl.ds(off, CHUNK), pl.ds(TAIL_START, TAIL)], sem_t[b])

    def wait_out(i, b):
        off = base + i * CHUNK
        pltpu.make_async_copy(
            rows_v[b].at[:, pl.ds(0, TAIL_START)],
            out_hbm.at[pl.ds(off, CHUNK), pl.ds(0, TAIL_START)],
            sem_b[b]).wait()
        pltpu.make_async_copy(
            tail_v[b],
            out_hbm.at[pl.ds(off, CHUNK), pl.ds(TAIL_START, TAIL)],
            sem_t[b]).wait()

    start_gather(0, 0)

    def body(g, c):
        for b in range(NBUF):
            i = g * NBUF + b

            @pl.when(i + 1 < N_CHUNKS)
            def _(i=i, nb=(b + 1) % NBUF):
                @pl.when(i >= 1)
                def _():
                    wait_out(i - 1, nb)
                start_gather(i + 1, nb)

            finish_chunk(i, b)
        return c

    lax.fori_loop(0, N_CHUNKS // NBUF, body, 0)
    wait_out(N_CHUNKS - 2, (N_CHUNKS - 2) % NBUF)
    wait_out(N_CHUNKS - 1, (N_CHUNKS - 1) % NBUF)


def kernel(inputs, embedding):
    idx = inputs.reshape(-1).astype(jnp.int32)
    table = jnp.pad(embedding, ((0, 0), (0, VOCAB_PAD - VOCAB)))
    out = _gather_kernel(idx, table)
    return out.reshape(inputs.shape[0], inputs.shape[1], VOCAB)
